# Initial kernel scaffold; baseline (speedup 1.0000x reference)
import functools

import jax
import jax.numpy as jnp
from jax import lax
from jax.experimental import pallas as pl
from jax.experimental.pallas import tpu as pltpu
from jax.experimental.pallas import tpu_sc as plsc

VOCAB = 1000000
EMB_DIM = 16
BATCH = 16384
HIST = 200

NC = 2
NS = 16
NW = NC * NS

ROWS_PER_TILE = BATCH // NW
ROWS_PER_CHUNK = 16
CHUNK = ROWS_PER_CHUNK * HIST
NCHUNKS = ROWS_PER_TILE // ROWS_PER_CHUNK


TC_BLOCK = 5000


def _score_body(e_ref, w_ref, b_ref, out_ref):
    lane = lax.broadcasted_iota(jnp.int32, (8, 128), 1)
    grp = lax.broadcasted_iota(jnp.int32, (8, 128), 0)
    w_tiled = jnp.tile(w_ref[:], (1, 8))
    wmt = jnp.where(lane // 16 == grp,
                    jnp.broadcast_to(w_tiled, (8, 128)), 0.0)
    s2 = lax.dot_general(e_ref[:], wmt, (((1,), (1,)), ((), ())),
                         preferred_element_type=jnp.float32)
    out_ref[:] = s2 * (1.0 / HIST) + b_ref[0, 0] * (1.0 / HIST)


def _score_table(emb2, w, b2):
    n = VOCAB // 8
    return pl.pallas_call(
        _score_body,
        grid=(n // TC_BLOCK,),
        in_specs=[
            pl.BlockSpec((TC_BLOCK, 128), lambda i: (i, 0)),
            pl.BlockSpec((1, EMB_DIM), lambda i: (0, 0)),
            pl.BlockSpec(memory_space=pltpu.SMEM),
        ],
        out_specs=pl.BlockSpec((TC_BLOCK, 8), lambda i: (i, 0)),
        out_shape=jax.ShapeDtypeStruct((n, 8), jnp.float32),
    )(emb2, w, b2)


_SEG = VOCAB // 8


def _sc_body(s_hbm, x_hbm, out_hbm, idx_v, vals_v, out_v, s_sp, sem):
    c = lax.axis_index("c")
    sid = lax.axis_index("s")
    wid = sid * NC + c

    @pl.when(sid < 8)
    def _():
        pltpu.sync_copy(s_hbm.at[pl.ds(sid * _SEG, _SEG)],
                        s_sp.at[pl.ds(sid * _SEG, _SEG)])
    plsc.subcore_barrier()

    row_lanes = lax.iota(jnp.int32, (16,)) * HIST
    tile_base = wid * ROWS_PER_TILE * HIST

    def chunk_body(g, carry):
        off = tile_base + g * CHUNK
        pltpu.sync_copy(x_hbm.at[pl.ds(off, CHUNK)], idx_v)
        pltpu.async_copy(s_sp.at[idx_v], vals_v, sem).wait()

        def jbody(j, acc):
            return acc + plsc.load_gather(vals_v, [row_lanes + j])

        acc = lax.fori_loop(0, HIST, jbody, jnp.zeros((16,), jnp.float32))
        y = 1.0 / (1.0 + jnp.exp(-acc))
        yq = (y * 10000.0 + 0.5).astype(jnp.int32).astype(jnp.float32) / 10000.0
        out_v[pl.ds(g * ROWS_PER_CHUNK, ROWS_PER_CHUNK)] = yq
        return carry

    lax.fori_loop(0, NCHUNKS, chunk_body, 0)
    pltpu.sync_copy(out_v, out_hbm.at[pl.ds(wid * ROWS_PER_TILE, ROWS_PER_TILE)])


@functools.partial(
    pl.kernel,
    out_type=jax.ShapeDtypeStruct((BATCH,), jnp.float32),
    mesh=plsc.VectorSubcoreMesh(core_axis_name="c", subcore_axis_name="s"),
    scratch_types=[
        pltpu.VMEM((CHUNK,), jnp.int32),
        pltpu.VMEM((CHUNK,), jnp.float32),
        pltpu.VMEM((ROWS_PER_TILE,), jnp.float32),
        pltpu.VMEM_SHARED((VOCAB,), jnp.float32),
        pltpu.SemaphoreType.DMA,
    ],
)
def _sc_pool(s_hbm, x_hbm, out_hbm, idx_v, vals_v, out_v, s_sp, sem):
    _sc_body(s_hbm, x_hbm, out_hbm, idx_v, vals_v, out_v, s_sp, sem)


def kernel(x, emb_table, W, b):
    emb2 = emb_table.reshape(VOCAB // 8, 128)
    s = _score_table(emb2, W, b.reshape(1, 1)).reshape(VOCAB)
    y = _sc_pool(s, x.reshape(BATCH * HIST))
    return y.reshape(BATCH, 1)


# speedup vs baseline: 8.9089x; 8.9089x over previous
import functools

import jax
import jax.numpy as jnp
from jax import lax
from jax.experimental import pallas as pl
from jax.experimental.pallas import tpu as pltpu
from jax.experimental.pallas import tpu_sc as plsc

VOCAB = 1000000
EMB_DIM = 16
BATCH = 16384
HIST = 200

NC = 2
NS = 16
NW = NC * NS

ROWS_PER_TILE = BATCH // NW
ROWS_PER_CHUNK = 16
CHUNK = ROWS_PER_CHUNK * HIST
NCHUNKS = ROWS_PER_TILE // ROWS_PER_CHUNK


TC_BLOCK = 5000


def _score_body(e_ref, w_ref, b_ref, out_ref):
    lane = lax.broadcasted_iota(jnp.int32, (8, 128), 1)
    grp = lax.broadcasted_iota(jnp.int32, (8, 128), 0)
    w_tiled = jnp.tile(w_ref[:], (1, 8))
    wmt = jnp.where(lane // 16 == grp,
                    jnp.broadcast_to(w_tiled, (8, 128)), 0.0)
    s2 = lax.dot_general(e_ref[:], wmt, (((1,), (1,)), ((), ())),
                         preferred_element_type=jnp.float32)
    out_ref[:] = s2 * (1.0 / HIST) + b_ref[0, 0] * (1.0 / HIST)


def _score_table(emb2, w, b2):
    n = VOCAB // 8
    return pl.pallas_call(
        _score_body,
        grid=(n // TC_BLOCK,),
        in_specs=[
            pl.BlockSpec((TC_BLOCK, 128), lambda i: (i, 0)),
            pl.BlockSpec((1, EMB_DIM), lambda i: (0, 0)),
            pl.BlockSpec(memory_space=pltpu.SMEM),
        ],
        out_specs=pl.BlockSpec((TC_BLOCK, 8), lambda i: (i, 0)),
        out_shape=jax.ShapeDtypeStruct((n, 8), jnp.float32),
    )(emb2, w, b2)


_SEG = VOCAB // 8
_PIECE = 25000
_NPIECE = _SEG // _PIECE


def _sc_body(s_hbm, x_hbm, out_hbm, idx_v, vals_v, out_v, stage_v, s_sp, sem):
    c = lax.axis_index("c")
    sid = lax.axis_index("s")
    wid = sid * NC + c

    @pl.when(sid < 8)
    def _():
        def stage_piece(k, carry):
            off = sid * _SEG + k * _PIECE
            pltpu.sync_copy(s_hbm.at[pl.ds(off, _PIECE)], stage_v)
            pltpu.sync_copy(stage_v, s_sp.at[pl.ds(off, _PIECE)])
            return carry

        lax.fori_loop(0, _NPIECE, stage_piece, 0)
    plsc.subcore_barrier()

    row_lanes = lax.iota(jnp.int32, 16) * HIST
    tile_base = wid * ROWS_PER_TILE * HIST

    def chunk_body(g, carry):
        off = tile_base + g * CHUNK
        pltpu.sync_copy(x_hbm.at[pl.ds(off, CHUNK)], idx_v)
        pltpu.async_copy(s_sp.at[idx_v], vals_v, sem).wait()

        def jbody(j, acc):
            return acc + plsc.load_gather(vals_v, [row_lanes + j])

        acc = lax.fori_loop(0, HIST, jbody, jnp.zeros((16,), jnp.float32))
        y = 1.0 / (1.0 + jnp.exp(-acc))
        yq = (y * 10000.0 + 0.5).astype(jnp.int32).astype(jnp.float32) / 10000.0
        out_v[pl.ds(g * ROWS_PER_CHUNK, ROWS_PER_CHUNK)] = yq
        return carry

    lax.fori_loop(0, NCHUNKS, chunk_body, 0)
    pltpu.sync_copy(out_v, out_hbm.at[pl.ds(wid * ROWS_PER_TILE, ROWS_PER_TILE)])


@functools.lru_cache(maxsize=None)
def _sc_pool():
    return pl.kernel(
        _sc_body,
        out_type=jax.ShapeDtypeStruct((BATCH,), jnp.float32),
        mesh=plsc.VectorSubcoreMesh(core_axis_name="c", subcore_axis_name="s",
                                    num_cores=NC, num_subcores=NS),
        compiler_params=pltpu.CompilerParams(needs_layout_passes=False),
        scratch_types=[
            pltpu.VMEM((CHUNK,), jnp.int32),
            pltpu.VMEM((CHUNK,), jnp.float32),
            pltpu.VMEM((ROWS_PER_TILE,), jnp.float32),
            pltpu.VMEM((_PIECE,), jnp.float32),
            pltpu.VMEM_SHARED((VOCAB,), jnp.float32),
            pltpu.SemaphoreType.DMA,
        ],
    )


def kernel(x, emb_table, W, b):
    emb2 = emb_table.reshape(VOCAB // 8, 128)
    s = _score_table(emb2, W, b.reshape(1, 1)).reshape(VOCAB)
    y = _sc_pool()(s, x.reshape(BATCH * HIST))
    return y.reshape(BATCH, 1)


# speedup vs baseline: 29.9835x; 3.3656x over previous
import functools

import jax
import jax.numpy as jnp
from jax import lax
from jax.experimental import pallas as pl
from jax.experimental.pallas import tpu as pltpu
from jax.experimental.pallas import tpu_sc as plsc

VOCAB = 1000000
EMB_DIM = 16
BATCH = 16384
HIST = 200

NC = 2
NS = 16
NW = NC * NS

ROWS_PER_TILE = BATCH // NW
ROWS_PER_CHUNK = 16
CHUNK = ROWS_PER_CHUNK * HIST
NCHUNKS = ROWS_PER_TILE // ROWS_PER_CHUNK


TC_BLOCK = 62464
TC_GRID = -(-VOCAB // TC_BLOCK)


def _score_body(e_ref, wt_ref, b_ref, out_ref):
    prod = e_ref[:] * wt_ref[:]
    s = jnp.sum(prod, axis=0, keepdims=True)
    row = pl.program_id(0) % 8
    out_ref[pl.ds(row, 1), :] = s * (1.0 / HIST) + b_ref[0, 0] * (1.0 / HIST)


def _score_table(emb_t, wt, b2):
    nrows = -(-TC_GRID // 8) * 8
    return pl.pallas_call(
        _score_body,
        grid=(TC_GRID,),
        in_specs=[
            pl.BlockSpec((EMB_DIM, TC_BLOCK), lambda i: (0, i)),
            pl.BlockSpec((EMB_DIM, 1), lambda i: (0, 0)),
            pl.BlockSpec(memory_space=pltpu.SMEM),
        ],
        out_specs=pl.BlockSpec((8, TC_BLOCK), lambda i: (i // 8, 0)),
        out_shape=jax.ShapeDtypeStruct((nrows, TC_BLOCK), jnp.float32),
    )(emb_t, wt, b2)


_SEG = VOCAB // 8
_PIECE = 25000
_NPIECE = _SEG // _PIECE


def _sc_body(s_hbm, x_hbm, out_hbm, idx_v, vals_v, out_v, stage_v, s_sp, sem):
    c = lax.axis_index("c")
    sid = lax.axis_index("s")
    wid = sid * NC + c

    @pl.when(sid < 8)
    def _():
        def stage_piece(k, carry):
            off = sid * _SEG + k * _PIECE
            pltpu.sync_copy(s_hbm.at[pl.ds(off, _PIECE)], stage_v)
            pltpu.sync_copy(stage_v, s_sp.at[pl.ds(off, _PIECE)])
            return carry

        lax.fori_loop(0, _NPIECE, stage_piece, 0)
    plsc.subcore_barrier()

    row_lanes = lax.iota(jnp.int32, 16) * HIST
    tile_base = wid * ROWS_PER_TILE * HIST

    def chunk_body(g, carry):
        off = tile_base + g * CHUNK
        pltpu.sync_copy(x_hbm.at[pl.ds(off, CHUNK)], idx_v)
        pltpu.async_copy(s_sp.at[idx_v], vals_v, sem).wait()

        def jbody(j, acc):
            return acc + plsc.load_gather(vals_v, [row_lanes + j])

        acc = lax.fori_loop(0, HIST, jbody, jnp.zeros((16,), jnp.float32))
        y = 1.0 / (1.0 + jnp.exp(-acc))
        yq = (y * 10000.0 + 0.5).astype(jnp.int32).astype(jnp.float32) / 10000.0
        out_v[pl.ds(g * ROWS_PER_CHUNK, ROWS_PER_CHUNK)] = yq
        return carry

    lax.fori_loop(0, NCHUNKS, chunk_body, 0)
    pltpu.sync_copy(out_v, out_hbm.at[pl.ds(wid * ROWS_PER_TILE, ROWS_PER_TILE)])


@functools.lru_cache(maxsize=None)
def _sc_pool():
    return pl.kernel(
        _sc_body,
        out_type=jax.ShapeDtypeStruct((BATCH,), jnp.float32),
        mesh=plsc.VectorSubcoreMesh(core_axis_name="c", subcore_axis_name="s",
                                    num_cores=NC, num_subcores=NS),
        compiler_params=pltpu.CompilerParams(needs_layout_passes=False),
        scratch_types=[
            pltpu.VMEM((CHUNK,), jnp.int32),
            pltpu.VMEM((CHUNK,), jnp.float32),
            pltpu.VMEM((ROWS_PER_TILE,), jnp.float32),
            pltpu.VMEM((_PIECE,), jnp.float32),
            pltpu.VMEM_SHARED((VOCAB,), jnp.float32),
            pltpu.SemaphoreType.DMA,
        ],
    )


def kernel(x, emb_table, W, b):
    emb_t = emb_table.T
    s = _score_table(emb_t, W.reshape(EMB_DIM, 1),
                     b.reshape(1, 1)).reshape(-1)[:VOCAB]
    y = _sc_pool()(s, x.reshape(BATCH * HIST))
    return y.reshape(BATCH, 1)


# speedup vs baseline: 41.6015x; 1.3875x over previous
import functools

import jax
import jax.numpy as jnp
from jax import lax
from jax.experimental import pallas as pl
from jax.experimental.pallas import tpu as pltpu
from jax.experimental.pallas import tpu_sc as plsc

VOCAB = 1000000
EMB_DIM = 16
BATCH = 16384
HIST = 200

NC = 2
NS = 16
NW = NC * NS

ROWS_PER_TILE = BATCH // NW
ROWS_PER_CHUNK = 16
CHUNK = ROWS_PER_CHUNK * HIST
NCHUNKS = ROWS_PER_TILE // ROWS_PER_CHUNK


TC_BLOCK = 62464
TC_GRID = -(-VOCAB // TC_BLOCK)


def _score_body(e_ref, wt_ref, b_ref, out_ref):
    prod = e_ref[:] * wt_ref[:]
    s = jnp.sum(prod, axis=0, keepdims=True)
    row = pl.program_id(0) % 8
    out_ref[pl.ds(row, 1), :] = s * (1.0 / HIST) + b_ref[0, 0] * (1.0 / HIST)


def _score_table(emb_t, wt, b2):
    nrows = -(-TC_GRID // 8) * 8
    return pl.pallas_call(
        _score_body,
        grid=(TC_GRID,),
        in_specs=[
            pl.BlockSpec((EMB_DIM, TC_BLOCK), lambda i: (0, i)),
            pl.BlockSpec((EMB_DIM, 1), lambda i: (0, 0)),
            pl.BlockSpec(memory_space=pltpu.SMEM),
        ],
        out_specs=pl.BlockSpec((8, TC_BLOCK), lambda i: (i // 8, 0)),
        out_shape=jax.ShapeDtypeStruct((nrows, TC_BLOCK), jnp.float32),
    )(emb_t, wt, b2)


_SEG = VOCAB // 8
_PIECE = 25000
_NPIECE = VOCAB // _PIECE

CHUNK_ROWS = 32
NCHUNK = ROWS_PER_TILE // CHUNK_ROWS
NGRP = CHUNK_ROWS // 16


def _sc_body(s_hbm, x_hbm, out_hbm, idx_v0, idx_v1, vals_v0, vals_v1, out_v,
             stage_v, s_sp, isem0, isem1, gsem0, gsem1):
    c = lax.axis_index("c")
    sid = lax.axis_index("s")
    wid = sid * NC + c
    row0 = wid * ROWS_PER_TILE
    idx_v = (idx_v0, idx_v1)
    vals_v = (vals_v0, vals_v1)
    isem = (isem0, isem1)
    gsem = (gsem0, gsem1)

    def start_idx(g, b):
        base = (row0 + g * CHUNK_ROWS) * HIST
        return pltpu.async_copy(
            x_hbm.at[pl.ds(base, CHUNK_ROWS * HIST)], idx_v[b], isem[b])

    def start_gather(b):
        return pltpu.async_copy(s_sp.at[idx_v[b]], vals_v[b], gsem[b])

    d_idx = [start_idx(0, 0), start_idx(1, 1)]

    def stage_piece(p, carry):
        off = p * _PIECE
        pltpu.sync_copy(s_hbm.at[pl.ds(off, _PIECE)], stage_v)
        pltpu.sync_copy(stage_v, s_sp.at[pl.ds(off, _PIECE)])
        return carry

    nmine = (_NPIECE - sid + NS - 1) // NS
    lax.fori_loop(0, nmine, lambda k, cr: stage_piece(k * NS + sid, cr), 0)
    plsc.subcore_barrier()

    lanes = lax.iota(jnp.int32, 16)

    def reduce_group(b, g16):
        rowbase = (g16 * 16 + lanes) * HIST

        def jbody(j, carry):
            acc, col = carry
            v = plsc.load_gather(vals_v[b], [rowbase + col])
            col = col + 1
            col = jnp.where(col >= HIST, col - HIST, col)
            return acc + v, col

        acc, _ = lax.fori_loop(
            0, HIST, jbody, (jnp.zeros((16,), jnp.float32), lanes))
        y = 1.0 / (1.0 + jnp.exp(-acc))
        return (y * 10000.0 + 0.5).astype(jnp.int32).astype(jnp.float32) / 10000.0

    d_gat = [None, None]
    d_idx[0].wait()
    d_gat[0] = start_gather(0)
    for g in range(NCHUNK):
        b = g & 1
        nb = 1 - b
        if g + 1 < NCHUNK:
            d_idx[nb].wait()
            d_gat[nb] = start_gather(nb)
        d_gat[b].wait()
        if g + 2 < NCHUNK:
            d_idx[b] = start_idx(g + 2, b)
        for g16 in range(NGRP):
            out_v[pl.ds(g * CHUNK_ROWS + g16 * 16, 16)] = reduce_group(b, g16)

    pltpu.sync_copy(out_v, out_hbm.at[pl.ds(row0, ROWS_PER_TILE)])


@functools.lru_cache(maxsize=None)
def _sc_pool():
    return pl.kernel(
        _sc_body,
        out_type=jax.ShapeDtypeStruct((BATCH,), jnp.float32),
        mesh=plsc.VectorSubcoreMesh(core_axis_name="c", subcore_axis_name="s",
                                    num_cores=NC, num_subcores=NS),
        compiler_params=pltpu.CompilerParams(needs_layout_passes=False),
        scratch_types=[
            pltpu.VMEM((CHUNK_ROWS * HIST,), jnp.int32),
            pltpu.VMEM((CHUNK_ROWS * HIST,), jnp.int32),
            pltpu.VMEM((CHUNK_ROWS * HIST,), jnp.float32),
            pltpu.VMEM((CHUNK_ROWS * HIST,), jnp.float32),
            pltpu.VMEM((ROWS_PER_TILE,), jnp.float32),
            pltpu.VMEM((_PIECE,), jnp.float32),
            pltpu.VMEM_SHARED((VOCAB,), jnp.float32),
            pltpu.SemaphoreType.DMA,
            pltpu.SemaphoreType.DMA,
            pltpu.SemaphoreType.DMA,
            pltpu.SemaphoreType.DMA,
        ],
    )


def kernel(x, emb_table, W, b):
    emb_t = emb_table.T
    s = _score_table(emb_t, W.reshape(EMB_DIM, 1),
                     b.reshape(1, 1)).reshape(-1)[:VOCAB]
    y = _sc_pool()(s, x.reshape(BATCH * HIST))
    return y.reshape(BATCH, 1)


# speedup vs baseline: 48.1226x; 1.1567x over previous
import functools

import jax
import jax.numpy as jnp
from jax import lax
from jax.experimental import pallas as pl
from jax.experimental.pallas import tpu as pltpu
from jax.experimental.pallas import tpu_sc as plsc

VOCAB = 1000000
EMB_DIM = 16
BATCH = 16384
HIST = 200

NC = 2
NS = 16
NW = NC * NS

ROWS_PER_TILE = BATCH // NW


TC_BLOCK = 62464
TC_GRID = -(-VOCAB // TC_BLOCK)


def _score_body(e_ref, wt_ref, b_ref, out_ref):
    prod = e_ref[:] * wt_ref[:]
    s = jnp.sum(prod, axis=0, keepdims=True)
    row = pl.program_id(0) % 8
    out_ref[pl.ds(row, 1), :] = s * (1.0 / HIST) + b_ref[0, 0] * (1.0 / HIST)


def _score_table(emb_t, wt, b2):
    nrows = -(-TC_GRID // 8) * 8
    return pl.pallas_call(
        _score_body,
        grid=(TC_GRID,),
        in_specs=[
            pl.BlockSpec((EMB_DIM, TC_BLOCK), lambda i: (0, i)),
            pl.BlockSpec((EMB_DIM, 1), lambda i: (0, 0)),
            pl.BlockSpec(memory_space=pltpu.SMEM),
        ],
        out_specs=pl.BlockSpec((8, TC_BLOCK), lambda i: (i // 8, 0)),
        out_shape=jax.ShapeDtypeStruct((nrows, TC_BLOCK), jnp.float32),
    )(emb_t, wt, b2)


_PIECE = 25000
_NPIECE = VOCAB // _PIECE

JB = 20
NJB = HIST // JB
BSZ = JB * ROWS_PER_TILE
NACC = ROWS_PER_TILE // 16


def _sc_body(s_hbm, x_hbm, out_hbm, idx_v0, idx_v1, vals_v0, vals_v1, out_v,
             stage_v, s_sp, isem0, isem1, gsem0, gsem1):
    c = lax.axis_index("c")
    sid = lax.axis_index("s")
    wid = sid * NC + c
    row0 = wid * ROWS_PER_TILE
    idx_v = (idx_v0, idx_v1)
    vals_v = (vals_v0, vals_v1)
    isem = (isem0, isem1)
    gsem = (gsem0, gsem1)

    def start_idx_batch(jb, b):
        return [
            pltpu.async_copy(
                x_hbm.at[pl.ds((jb * JB + jj) * BATCH + row0, ROWS_PER_TILE)],
                idx_v[b].at[pl.ds(jj * ROWS_PER_TILE, ROWS_PER_TILE)],
                isem[b])
            for jj in range(JB)
        ]

    def start_gather(b):
        return pltpu.async_copy(s_sp.at[idx_v[b]], vals_v[b], gsem[b])

    d_idx = [start_idx_batch(0, 0), start_idx_batch(1, 1)]

    def stage_piece(p, carry):
        off = p * _PIECE
        pltpu.sync_copy(s_hbm.at[pl.ds(off, _PIECE)], stage_v)
        pltpu.sync_copy(stage_v, s_sp.at[pl.ds(off, _PIECE)])
        return carry

    nmine = (_NPIECE - sid + NS - 1) // NS
    lax.fori_loop(0, nmine, lambda k, cr: stage_piece(k * NS + sid, cr), 0)
    plsc.subcore_barrier()

    acc = [jnp.zeros((16,), jnp.float32) for _ in range(NACC)]

    def reduce_batch(b, acc):
        def jbody(jj, acc):
            base = jj * ROWS_PER_TILE
            return tuple(
                acc[k] + vals_v[b][pl.ds(base + k * 16, 16)]
                for k in range(NACC))

        return list(lax.fori_loop(0, JB, jbody, tuple(acc)))

    d_gat = [None, None]
    for d in d_idx[0]:
        d.wait()
    d_gat[0] = start_gather(0)
    for gb in range(NJB):
        b = gb & 1
        nb = 1 - b
        if gb + 1 < NJB:
            for d in d_idx[nb]:
                d.wait()
            d_gat[nb] = start_gather(nb)
        d_gat[b].wait()
        if gb + 2 < NJB:
            d_idx[b] = start_idx_batch(gb + 2, b)
        acc = reduce_batch(b, acc)

    for k in range(NACC):
        y = 1.0 / (1.0 + jnp.exp(-acc[k]))
        yq = (y * 10000.0 + 0.5).astype(jnp.int32).astype(jnp.float32) / 10000.0
        out_v[pl.ds(k * 16, 16)] = yq
    pltpu.sync_copy(out_v, out_hbm.at[pl.ds(row0, ROWS_PER_TILE)])


@functools.lru_cache(maxsize=None)
def _sc_pool():
    return pl.kernel(
        _sc_body,
        out_type=jax.ShapeDtypeStruct((BATCH,), jnp.float32),
        mesh=plsc.VectorSubcoreMesh(core_axis_name="c", subcore_axis_name="s",
                                    num_cores=NC, num_subcores=NS),
        compiler_params=pltpu.CompilerParams(needs_layout_passes=False),
        scratch_types=[
            pltpu.VMEM((BSZ,), jnp.int32),
            pltpu.VMEM((BSZ,), jnp.int32),
            pltpu.VMEM((BSZ,), jnp.float32),
            pltpu.VMEM((BSZ,), jnp.float32),
            pltpu.VMEM((ROWS_PER_TILE,), jnp.float32),
            pltpu.VMEM((_PIECE,), jnp.float32),
            pltpu.VMEM_SHARED((VOCAB,), jnp.float32),
            pltpu.SemaphoreType.DMA,
            pltpu.SemaphoreType.DMA,
            pltpu.SemaphoreType.DMA,
            pltpu.SemaphoreType.DMA,
        ],
    )


def kernel(x, emb_table, W, b):
    emb_t = emb_table.T
    s = _score_table(emb_t, W.reshape(EMB_DIM, 1),
                     b.reshape(1, 1)).reshape(-1)[:VOCAB]
    xt = x.T.reshape(HIST * BATCH)
    y = _sc_pool()(s, xt)
    return y.reshape(BATCH, 1)


# speedup vs baseline: 48.5714x; 1.0093x over previous
import functools

import jax
import jax.numpy as jnp
from jax import lax
from jax.experimental import pallas as pl
from jax.experimental.pallas import tpu as pltpu
from jax.experimental.pallas import tpu_sc as plsc

VOCAB = 1000000
EMB_DIM = 16
BATCH = 16384
HIST = 200

NC = 2
NS = 16
NW = NC * NS

ROWS_PER_TILE = BATCH // NW


TC_BLOCK = 62464
TC_GRID = -(-VOCAB // TC_BLOCK)


def _score_body(e_ref, wt_ref, b_ref, out_ref):
    prod = e_ref[:] * wt_ref[:]
    s = jnp.sum(prod, axis=0, keepdims=True)
    row = pl.program_id(0) % 8
    out_ref[pl.ds(row, 1), :] = s * (1.0 / HIST) + b_ref[0, 0] * (1.0 / HIST)


S_ROWS = -(-TC_GRID // 8) * 8
S_PAD = S_ROWS * TC_BLOCK


def _score_table(emb_t, wt, b2):
    return pl.pallas_call(
        _score_body,
        grid=(TC_GRID,),
        in_specs=[
            pl.BlockSpec((EMB_DIM, TC_BLOCK), lambda i: (0, i)),
            pl.BlockSpec((EMB_DIM, 1), lambda i: (0, 0)),
            pl.BlockSpec(memory_space=pltpu.SMEM),
        ],
        out_specs=pl.BlockSpec((8, TC_BLOCK), lambda i: (i // 8, 0)),
        out_shape=jax.ShapeDtypeStruct((S_ROWS, TC_BLOCK), jnp.float32),
    )(emb_t, wt, b2)


_PIECE = 25000
_NPIECE = VOCAB // _PIECE

JB = 20
NJB = HIST // JB
BSZ = JB * ROWS_PER_TILE
NACC = ROWS_PER_TILE // 16


def _sc_body(s_hbm, x_hbm, out_hbm, idx_v0, idx_v1, vals_v0, vals_v1, out_v,
             stage_v, s_sp, isem0, isem1, gsem0, gsem1):
    c = lax.axis_index("c")
    sid = lax.axis_index("s")
    wid = sid * NC + c
    row0 = wid * ROWS_PER_TILE
    idx_v = (idx_v0, idx_v1)
    vals_v = (vals_v0, vals_v1)
    isem = (isem0, isem1)
    gsem = (gsem0, gsem1)

    def start_idx_batch(jb, b):
        return [
            pltpu.async_copy(
                x_hbm.at[pl.ds((jb * JB + jj) * BATCH + row0, ROWS_PER_TILE)],
                idx_v[b].at[pl.ds(jj * ROWS_PER_TILE, ROWS_PER_TILE)],
                isem[b])
            for jj in range(JB)
        ]

    def start_gather(b):
        return pltpu.async_copy(s_sp.at[idx_v[b]], vals_v[b], gsem[b])

    d_idx = [start_idx_batch(0, 0), start_idx_batch(1, 1)]

    def stage_piece(p, carry):
        off = p * _PIECE
        pltpu.sync_copy(s_hbm.at[pl.ds(off, _PIECE)], stage_v)
        pltpu.sync_copy(stage_v, s_sp.at[pl.ds(off, _PIECE)])
        return carry

    nmine = (_NPIECE - sid + NS - 1) // NS
    lax.fori_loop(0, nmine, lambda k, cr: stage_piece(k * NS + sid, cr), 0)
    plsc.subcore_barrier()

    acc = [jnp.zeros((16,), jnp.float32) for _ in range(NACC)]

    def reduce_batch(b, acc):
        def jbody(jj, acc):
            base = jj * ROWS_PER_TILE
            return tuple(
                acc[k] + vals_v[b][pl.ds(base + k * 16, 16)]
                for k in range(NACC))

        return list(lax.fori_loop(0, JB, jbody, tuple(acc)))

    d_gat = [None, None]
    for d in d_idx[0]:
        d.wait()
    d_gat[0] = start_gather(0)
    for gb in range(NJB):
        b = gb & 1
        nb = 1 - b
        if gb + 1 < NJB:
            for d in d_idx[nb]:
                d.wait()
            d_gat[nb] = start_gather(nb)
        d_gat[b].wait()
        if gb + 2 < NJB:
            d_idx[b] = start_idx_batch(gb + 2, b)
        acc = reduce_batch(b, acc)

    for k in range(NACC):
        y = 1.0 / (1.0 + jnp.exp(-acc[k]))
        yq = (y * 10000.0 + 0.5).astype(jnp.int32).astype(jnp.float32) / 10000.0
        out_v[pl.ds(k * 16, 16)] = yq
    pltpu.sync_copy(out_v, out_hbm.at[pl.ds(row0, ROWS_PER_TILE)])


@functools.lru_cache(maxsize=None)
def _sc_pool():
    return pl.kernel(
        _sc_body,
        out_type=jax.ShapeDtypeStruct((BATCH,), jnp.float32),
        mesh=plsc.VectorSubcoreMesh(core_axis_name="c", subcore_axis_name="s",
                                    num_cores=NC, num_subcores=NS),
        compiler_params=pltpu.CompilerParams(needs_layout_passes=False),
        scratch_types=[
            pltpu.VMEM((BSZ,), jnp.int32),
            pltpu.VMEM((BSZ,), jnp.int32),
            pltpu.VMEM((BSZ,), jnp.float32),
            pltpu.VMEM((BSZ,), jnp.float32),
            pltpu.VMEM((ROWS_PER_TILE,), jnp.float32),
            pltpu.VMEM((_PIECE,), jnp.float32),
            pltpu.VMEM_SHARED((VOCAB,), jnp.float32),
            pltpu.SemaphoreType.DMA,
            pltpu.SemaphoreType.DMA,
            pltpu.SemaphoreType.DMA,
            pltpu.SemaphoreType.DMA,
        ],
    )


def kernel(x, emb_table, W, b):
    xt = x.T.reshape(HIST * BATCH)
    emb_t = emb_table.T
    s_pad = _score_table(emb_t, W.reshape(EMB_DIM, 1),
                         b.reshape(1, 1)).reshape(S_PAD)
    y = _sc_pool()(s_pad, xt)
    return y.reshape(BATCH, 1)


# speedup vs baseline: 48.6127x; 1.0008x over previous
import functools

import jax
import jax.numpy as jnp
from jax import lax
from jax.experimental import pallas as pl
from jax.experimental.pallas import tpu as pltpu
from jax.experimental.pallas import tpu_sc as plsc

VOCAB = 1000000
EMB_DIM = 16
BATCH = 16384
HIST = 200

NC = 2
NS = 16
NW = NC * NS

ROWS_PER_TILE = BATCH // NW


TC_BLOCK = 62464
TC_GRID = -(-VOCAB // TC_BLOCK)


def _score_body(e_ref, wt_ref, b_ref, out_ref):
    prod = e_ref[:] * wt_ref[:]
    s = jnp.sum(prod, axis=0, keepdims=True)
    row = pl.program_id(0) % 8
    out_ref[pl.ds(row, 1), :] = s * (1.0 / HIST) + b_ref[0, 0] * (1.0 / HIST)


S_ROWS = -(-TC_GRID // 8) * 8
S_PAD = S_ROWS * TC_BLOCK


def _score_table(emb_t, wt, b2):
    return pl.pallas_call(
        _score_body,
        grid=(TC_GRID,),
        in_specs=[
            pl.BlockSpec((EMB_DIM, TC_BLOCK), lambda i: (0, i)),
            pl.BlockSpec((EMB_DIM, 1), lambda i: (0, 0)),
            pl.BlockSpec(memory_space=pltpu.SMEM),
        ],
        out_specs=pl.BlockSpec((8, TC_BLOCK), lambda i: (i // 8, 0)),
        out_shape=jax.ShapeDtypeStruct((S_ROWS, TC_BLOCK), jnp.float32),
    )(emb_t, wt, b2)


_PIECE = 25000
_NPIECE = VOCAB // _PIECE

JB = 20
NJB = HIST // JB
BSZ = JB * ROWS_PER_TILE
NACC = ROWS_PER_TILE // 16


def _sc_body(s_hbm, x_hbm, out_hbm, idx_v0, idx_v1, vals_v0, vals_v1, out_v,
             stage_v, s_sp, isem0, isem1, gsem0, gsem1):
    c = lax.axis_index("c")
    sid = lax.axis_index("s")
    wid = sid * NC + c
    row0 = wid * ROWS_PER_TILE
    idx_v = (idx_v0, idx_v1)
    vals_v = (vals_v0, vals_v1)
    isem = (isem0, isem1)
    gsem = (gsem0, gsem1)

    def start_idx_batch(jb, b):
        return [
            pltpu.async_copy(
                x_hbm.at[pl.ds((jb * JB + jj) * BATCH + row0, ROWS_PER_TILE)],
                idx_v[b].at[pl.ds(jj * ROWS_PER_TILE, ROWS_PER_TILE)],
                isem[b])
            for jj in range(JB)
        ]

    def start_gather(b):
        h = BSZ // 2
        return [
            pltpu.async_copy(s_sp.at[idx_v[b].at[pl.ds(0, h)]],
                             vals_v[b].at[pl.ds(0, h)], gsem[b]),
            pltpu.async_copy(s_sp.at[idx_v[b].at[pl.ds(h, h)]],
                             vals_v[b].at[pl.ds(h, h)], gsem[b]),
        ]

    d_idx = [start_idx_batch(0, 0), start_idx_batch(1, 1)]

    def stage_piece(p, carry):
        off = p * _PIECE
        pltpu.sync_copy(s_hbm.at[pl.ds(off, _PIECE)], stage_v)
        pltpu.sync_copy(stage_v, s_sp.at[pl.ds(off, _PIECE)])
        return carry

    nmine = (_NPIECE - sid + NS - 1) // NS
    lax.fori_loop(0, nmine, lambda k, cr: stage_piece(k * NS + sid, cr), 0)
    plsc.subcore_barrier()

    acc = [jnp.zeros((16,), jnp.float32) for _ in range(NACC)]

    def reduce_batch(b, acc):
        def jbody(jj, acc):
            base = jj * ROWS_PER_TILE
            return tuple(
                acc[k] + vals_v[b][pl.ds(base + k * 16, 16)]
                for k in range(NACC))

        return list(lax.fori_loop(0, JB, jbody, tuple(acc)))

    d_gat = [None, None]
    for d in d_idx[0]:
        d.wait()
    d_gat[0] = start_gather(0)
    for gb in range(NJB):
        b = gb & 1
        nb = 1 - b
        if gb + 1 < NJB:
            for d in d_idx[nb]:
                d.wait()
            d_gat[nb] = start_gather(nb)
        for d in d_gat[b]:
            d.wait()
        if gb + 2 < NJB:
            d_idx[b] = start_idx_batch(gb + 2, b)
        acc = reduce_batch(b, acc)

    for k in range(NACC):
        y = 1.0 / (1.0 + jnp.exp(-acc[k]))
        yq = (y * 10000.0 + 0.5).astype(jnp.int32).astype(jnp.float32) / 10000.0
        out_v[pl.ds(k * 16, 16)] = yq
    pltpu.sync_copy(out_v, out_hbm.at[pl.ds(row0, ROWS_PER_TILE)])


@functools.lru_cache(maxsize=None)
def _sc_pool():
    return pl.kernel(
        _sc_body,
        out_type=jax.ShapeDtypeStruct((BATCH,), jnp.float32),
        mesh=plsc.VectorSubcoreMesh(core_axis_name="c", subcore_axis_name="s",
                                    num_cores=NC, num_subcores=NS),
        compiler_params=pltpu.CompilerParams(needs_layout_passes=False),
        scratch_types=[
            pltpu.VMEM((BSZ,), jnp.int32),
            pltpu.VMEM((BSZ,), jnp.int32),
            pltpu.VMEM((BSZ,), jnp.float32),
            pltpu.VMEM((BSZ,), jnp.float32),
            pltpu.VMEM((ROWS_PER_TILE,), jnp.float32),
            pltpu.VMEM((_PIECE,), jnp.float32),
            pltpu.VMEM_SHARED((VOCAB,), jnp.float32),
            pltpu.SemaphoreType.DMA,
            pltpu.SemaphoreType.DMA,
            pltpu.SemaphoreType.DMA,
            pltpu.SemaphoreType.DMA,
        ],
    )


def kernel(x, emb_table, W, b):
    xt = x.T.reshape(HIST * BATCH)
    emb_t = emb_table.T
    s_pad = _score_table(emb_t, W.reshape(EMB_DIM, 1),
                         b.reshape(1, 1)).reshape(S_PAD)
    y = _sc_pool()(s_pad, xt)
    return y.reshape(BATCH, 1)


# speedup vs baseline: 58.8892x; 1.2114x over previous
import functools

import jax
import jax.numpy as jnp
from jax import lax
from jax.experimental import pallas as pl
from jax.experimental.pallas import tpu as pltpu
from jax.experimental.pallas import tpu_sc as plsc

VOCAB = 1000000
EMB_DIM = 16
BATCH = 16384
HIST = 200

NC = 2
NS = 16
NW = NC * NS

ROWS_PER_TILE = BATCH // NW


TC_BLOCK = 62464
TC_GRID = -(-VOCAB // TC_BLOCK)


def _score_body(e_ref, wt_ref, b_ref, out_ref):
    prod = e_ref[:] * wt_ref[:]
    s = jnp.sum(prod, axis=0)
    out_ref[:] = s * (1.0 / HIST) + b_ref[0, 0] * (1.0 / HIST)


S_PAD = TC_GRID * TC_BLOCK


def _score_table(emb_t, wt, b2):
    return pl.pallas_call(
        _score_body,
        grid=(TC_GRID,),
        in_specs=[
            pl.BlockSpec((EMB_DIM, TC_BLOCK), lambda i: (0, i)),
            pl.BlockSpec((EMB_DIM, 1), lambda i: (0, 0)),
            pl.BlockSpec(memory_space=pltpu.SMEM),
        ],
        out_specs=pl.BlockSpec((TC_BLOCK,), lambda i: (i,)),
        out_shape=jax.ShapeDtypeStruct((S_PAD,), jnp.float32),
    )(emb_t, wt, b2)


_PIECE = 25000
_NPIECE = VOCAB // _PIECE

JB = 20
NJB = HIST // JB
BSZ = JB * ROWS_PER_TILE
NACC = ROWS_PER_TILE // 16


def _sc_body(s_hbm, x_hbm, out_hbm, idx_v0, idx_v1, vals_v0, vals_v1, out_v,
             stage_v, s_sp, isem0, isem1, gsem0, gsem1):
    c = lax.axis_index("c")
    sid = lax.axis_index("s")
    wid = sid * NC + c
    row0 = wid * ROWS_PER_TILE
    idx_v = (idx_v0, idx_v1)
    vals_v = (vals_v0, vals_v1)
    isem = (isem0, isem1)
    gsem = (gsem0, gsem1)

    def start_idx_batch(jb, b):
        return [
            pltpu.async_copy(
                x_hbm.at[pl.ds((jb * JB + jj) * BATCH + row0, ROWS_PER_TILE)],
                idx_v[b].at[pl.ds(jj * ROWS_PER_TILE, ROWS_PER_TILE)],
                isem[b])
            for jj in range(JB)
        ]

    def start_gather(b):
        h = BSZ // 2
        return [
            pltpu.async_copy(s_sp.at[idx_v[b].at[pl.ds(0, h)]],
                             vals_v[b].at[pl.ds(0, h)], gsem[b]),
            pltpu.async_copy(s_sp.at[idx_v[b].at[pl.ds(h, h)]],
                             vals_v[b].at[pl.ds(h, h)], gsem[b]),
        ]

    d_idx = [start_idx_batch(0, 0), start_idx_batch(1, 1)]

    def stage_piece(p, carry):
        off = p * _PIECE
        pltpu.sync_copy(s_hbm.at[pl.ds(off, _PIECE)], stage_v)
        pltpu.sync_copy(stage_v, s_sp.at[pl.ds(off, _PIECE)])
        return carry

    nmine = (_NPIECE - sid + NS - 1) // NS
    lax.fori_loop(0, nmine, lambda k, cr: stage_piece(k * NS + sid, cr), 0)
    plsc.subcore_barrier()

    acc = [jnp.zeros((16,), jnp.float32) for _ in range(NACC)]

    def reduce_batch(b, acc):
        def jbody(jj, acc):
            base = jj * ROWS_PER_TILE
            return tuple(
                acc[k] + vals_v[b][pl.ds(base + k * 16, 16)]
                for k in range(NACC))

        return list(lax.fori_loop(0, JB, jbody, tuple(acc)))

    d_gat = [None, None]
    for d in d_idx[0]:
        d.wait()
    d_gat[0] = start_gather(0)
    for gb in range(NJB):
        b = gb & 1
        nb = 1 - b
        if gb + 1 < NJB:
            for d in d_idx[nb]:
                d.wait()
            d_gat[nb] = start_gather(nb)
        for d in d_gat[b]:
            d.wait()
        if gb + 2 < NJB:
            d_idx[b] = start_idx_batch(gb + 2, b)
        acc = reduce_batch(b, acc)

    for k in range(NACC):
        y = 1.0 / (1.0 + jnp.exp(-acc[k]))
        yq = (y * 10000.0 + 0.5).astype(jnp.int32).astype(jnp.float32) / 10000.0
        out_v[pl.ds(k * 16, 16)] = yq
    pltpu.sync_copy(out_v, out_hbm.at[pl.ds(row0, ROWS_PER_TILE)])


@functools.lru_cache(maxsize=None)
def _sc_pool():
    return pl.kernel(
        _sc_body,
        out_type=jax.ShapeDtypeStruct((BATCH,), jnp.float32),
        mesh=plsc.VectorSubcoreMesh(core_axis_name="c", subcore_axis_name="s",
                                    num_cores=NC, num_subcores=NS),
        compiler_params=pltpu.CompilerParams(needs_layout_passes=False),
        scratch_types=[
            pltpu.VMEM((BSZ,), jnp.int32),
            pltpu.VMEM((BSZ,), jnp.int32),
            pltpu.VMEM((BSZ,), jnp.float32),
            pltpu.VMEM((BSZ,), jnp.float32),
            pltpu.VMEM((ROWS_PER_TILE,), jnp.float32),
            pltpu.VMEM((_PIECE,), jnp.float32),
            pltpu.VMEM_SHARED((VOCAB,), jnp.float32),
            pltpu.SemaphoreType.DMA,
            pltpu.SemaphoreType.DMA,
            pltpu.SemaphoreType.DMA,
            pltpu.SemaphoreType.DMA,
        ],
    )


def kernel(x, emb_table, W, b):
    xt = x.T.reshape(HIST * BATCH)
    emb_t = emb_table.T
    s_pad = _score_table(emb_t, W.reshape(EMB_DIM, 1), b.reshape(1, 1))
    y = _sc_pool()(s_pad, xt)
    return y.reshape(BATCH, 1)
